# Initial kernel scaffold; baseline (speedup 1.0000x reference)
import jax
import jax.numpy as jnp
from jax.experimental import pallas as pl


def kernel(attn, w_noise):
    raise NotImplementedError("write your pallas kernel here")


# speedup vs baseline: 29.6255x; 29.6255x over previous
import functools

import jax
import jax.numpy as jnp
from jax import lax
from jax.experimental import pallas as pl

_N = 2048
_K = 8
_SCALE = 1.0 / (2048.0 ** 0.5)
_ROWS_PER_BLOCK = 256


def _topk_body(x_ref, o_ref):
    x = x_ref[...]
    lanes = lax.broadcasted_iota(jnp.int32, x.shape, 1)
    neg_inf = jnp.float32(float("-inf"))
    m1 = jnp.max(x, axis=-1, keepdims=True)
    out = jnp.zeros_like(x)
    denom = jnp.zeros_like(m1)
    xw = x
    for _ in range(_K):
        mi = jnp.max(xw, axis=-1, keepdims=True)
        eq = xw == mi
        amin = jnp.min(jnp.where(eq, lanes, _N), axis=-1, keepdims=True)
        sel = lanes == amin
        e = jnp.exp((mi - m1) * _SCALE)
        out = jnp.where(sel, e, out)
        denom = denom + e
        xw = jnp.where(sel, neg_inf, xw)
    o_ref[...] = out / denom


@functools.partial(jax.jit, static_argnames=())
def kernel(attn, w_noise):
    del w_noise
    b, s, n = attn.shape
    rows = b * s
    x = attn.reshape(rows, n)
    grid = rows // _ROWS_PER_BLOCK
    out = pl.pallas_call(
        _topk_body,
        grid=(grid,),
        in_specs=[pl.BlockSpec((_ROWS_PER_BLOCK, n), lambda i: (i, 0))],
        out_specs=pl.BlockSpec((_ROWS_PER_BLOCK, n), lambda i: (i, 0)),
        out_shape=jax.ShapeDtypeStruct((rows, n), jnp.float32),
    )(x)
    return out.reshape(b, s, n)
